# baseline (device time: 163428 ns/iter reference)
import jax
import jax.numpy as jnp
from jax import lax
from jax.experimental import pallas as pl
from jax.experimental.pallas import tpu as pltpu

N_ROWS = 4096
N_COLS = 1024


def kernel(x, dest):
    zi = lax.axis_index("z")
    keep = dest == zi
    ck = jnp.sum(keep.astype(jnp.int32))

    order = jnp.argsort(jnp.logical_not(keep), stable=True)
    matA = x.astype(jnp.bfloat16)[order]
    rot = jnp.roll(matA, -ck, axis=0)

    is0 = zi == 0
    lo = jnp.where(is0, 0, N_ROWS - ck).astype(jnp.int32)
    hi = jnp.where(is0, ck, N_ROWS).astype(jnp.int32)
    params = jnp.stack([lo, hi]).astype(jnp.int32)

    def body(params_ref, matA_ref, rot_ref, out_ref, comm_ref, send_sem, recv_sem):
        my_x = lax.axis_index("x")
        my_y = lax.axis_index("y")
        my_z = lax.axis_index("z")
        nbr = (my_x, my_y, 1 - my_z)

        barrier_sem = pltpu.get_barrier_semaphore()
        pl.semaphore_signal(
            barrier_sem, inc=1, device_id=nbr, device_id_type=pl.DeviceIdType.MESH
        )
        pl.semaphore_wait(barrier_sem, 1)

        @pl.when(my_z == 0)
        def _():
            rdma = pltpu.make_async_remote_copy(
                src_ref=rot_ref,
                dst_ref=comm_ref,
                send_sem=send_sem,
                recv_sem=recv_sem,
                device_id=nbr,
                device_id_type=pl.DeviceIdType.MESH,
            )
            rdma.start()
            rdma.wait()

        @pl.when(my_z == 1)
        def _():
            rdma = pltpu.make_async_remote_copy(
                src_ref=matA_ref,
                dst_ref=comm_ref,
                send_sem=send_sem,
                recv_sem=recv_sem,
                device_id=nbr,
                device_id_type=pl.DeviceIdType.MESH,
            )
            rdma.start()
            rdma.wait()

        lo_s = params_ref[0]
        hi_s = params_ref[1]
        rows = lax.broadcasted_iota(jnp.int32, (N_ROWS, N_COLS), 0)
        keep_vals = jnp.where(my_z == 0, matA_ref[...], rot_ref[...])
        mask = (rows >= lo_s) & (rows < hi_s)
        out_ref[...] = jnp.where(mask, keep_vals, comm_ref[...])

    return pl.pallas_call(
        body,
        out_shape=jax.ShapeDtypeStruct((N_ROWS, N_COLS), jnp.bfloat16),
        in_specs=[
            pl.BlockSpec(memory_space=pltpu.SMEM),
            pl.BlockSpec(memory_space=pltpu.VMEM),
            pl.BlockSpec(memory_space=pltpu.VMEM),
        ],
        out_specs=pl.BlockSpec(memory_space=pltpu.VMEM),
        scratch_shapes=[
            pltpu.VMEM((N_ROWS, N_COLS), jnp.bfloat16),
            pltpu.SemaphoreType.DMA,
            pltpu.SemaphoreType.DMA,
        ],
        compiler_params=pltpu.CompilerParams(collective_id=0),
    )(params, matA, rot)


# device time: 118393 ns/iter; 1.3804x vs baseline; 1.3804x over previous
import jax
import jax.numpy as jnp
from jax import lax
from jax.experimental import pallas as pl
from jax.experimental.pallas import tpu as pltpu

N_ROWS = 4096
N_COLS = 1024
BLK = 512
MAX_BLK = N_ROWS // BLK


def kernel(x, dest):
    zi = lax.axis_index("z")
    keep = dest == zi
    ck = jnp.sum(keep.astype(jnp.int32))
    cs = N_ROWS - ck

    order = jnp.argsort(jnp.logical_not(keep), stable=True)
    matA = x.astype(jnp.bfloat16)[order]
    rot = jnp.roll(matA, -ck, axis=0)

    is0 = zi == 0
    lo = jnp.where(is0, 0, cs).astype(jnp.int32)
    hi = jnp.where(is0, ck, N_ROWS).astype(jnp.int32)
    nb = (cs + BLK - 1) // BLK
    params = jnp.stack([lo, hi, nb]).astype(jnp.int32)

    def body(params_ref, matA_ref, rot_ref, out_ref, comm_ref, send_sems, recv_sems):
        my_x = lax.axis_index("x")
        my_y = lax.axis_index("y")
        my_z = lax.axis_index("z")
        nbr = (my_x, my_y, 1 - my_z)

        lo_s = params_ref[0]
        hi_s = params_ref[1]
        nb_s = params_ref[2]

        barrier_sem = pltpu.get_barrier_semaphore()
        pl.semaphore_signal(
            barrier_sem, inc=1, device_id=nbr, device_id_type=pl.DeviceIdType.MESH
        )
        pl.semaphore_wait(barrier_sem, 1)

        def send_desc(i, src):
            return pltpu.make_async_remote_copy(
                src_ref=src.at[pl.ds(i * BLK, BLK)],
                dst_ref=comm_ref.at[pl.ds(i * BLK, BLK)],
                send_sem=send_sems.at[i],
                recv_sem=recv_sems.at[i],
                device_id=nbr,
                device_id_type=pl.DeviceIdType.MESH,
            )

        for i in range(MAX_BLK):
            @pl.when(jnp.logical_and(my_z == 0, i < nb_s))
            def _():
                send_desc(i, rot_ref).start()

            @pl.when(jnp.logical_and(my_z == 1, MAX_BLK - i <= nb_s))
            def _():
                send_desc(i, matA_ref).start()

        for i in range(MAX_BLK):
            @pl.when(jnp.logical_or(
                jnp.logical_and(my_z == 0, i < nb_s),
                jnp.logical_and(my_z == 1, MAX_BLK - i <= nb_s),
            ))
            def _():
                send_desc(i, matA_ref).wait_send()

        for i in range(MAX_BLK):
            @pl.when(jnp.logical_or(
                jnp.logical_and(my_z == 1, i < nb_s),
                jnp.logical_and(my_z == 0, MAX_BLK - i <= nb_s),
            ))
            def _():
                send_desc(i, matA_ref).wait_recv()

        rows = lax.broadcasted_iota(jnp.int32, (N_ROWS, N_COLS), 0)
        keep_vals = jnp.where(my_z == 0, matA_ref[...], rot_ref[...])
        mask = (rows >= lo_s) & (rows < hi_s)
        out_ref[...] = jnp.where(mask, keep_vals, comm_ref[...])

    return pl.pallas_call(
        body,
        out_shape=jax.ShapeDtypeStruct((N_ROWS, N_COLS), jnp.bfloat16),
        in_specs=[
            pl.BlockSpec(memory_space=pltpu.SMEM),
            pl.BlockSpec(memory_space=pltpu.VMEM),
            pl.BlockSpec(memory_space=pltpu.VMEM),
        ],
        out_specs=pl.BlockSpec(memory_space=pltpu.VMEM),
        scratch_shapes=[
            pltpu.VMEM((N_ROWS, N_COLS), jnp.bfloat16),
            pltpu.SemaphoreType.DMA((MAX_BLK,)),
            pltpu.SemaphoreType.DMA((MAX_BLK,)),
        ],
        compiler_params=pltpu.CompilerParams(collective_id=0),
    )(params, matA, rot)


# device time: 90792 ns/iter; 1.8000x vs baseline; 1.3040x over previous
import jax
import jax.numpy as jnp
from jax import lax
from jax.experimental import pallas as pl
from jax.experimental.pallas import tpu as pltpu

N_ROWS = 4096
N_COLS = 1024
BLK = 512
MAX_BLK = N_ROWS // BLK


def kernel(x, dest):
    zi = lax.axis_index("z")
    keep = dest == zi
    ck = jnp.sum(keep.astype(jnp.int32))
    cs = N_ROWS - ck

    order = jnp.argsort(jnp.logical_not(keep), stable=True)
    matA = x.astype(jnp.bfloat16)[order].reshape(N_ROWS, 8, 128)

    nb = (cs + BLK - 1) // BLK
    nbk = (ck + BLK - 1) // BLK
    params = jnp.stack([ck, cs, nb, nbk]).astype(jnp.int32)

    def body(params_ref, matA_ref, out_ref, send_sems, recv_sems, copy_sems):
        my_x = lax.axis_index("x")
        my_y = lax.axis_index("y")
        my_z = lax.axis_index("z")
        nbr = (my_x, my_y, 1 - my_z)

        ck_s = params_ref[0]
        cs_s = params_ref[1]
        nb_s = params_ref[2]
        nbk_s = params_ref[3]
        shift = ck_s * (1 - my_z)
        off = cs_s * my_z

        barrier_sem = pltpu.get_barrier_semaphore()
        pl.semaphore_signal(
            barrier_sem, inc=1, device_id=nbr, device_id_type=pl.DeviceIdType.MESH
        )
        pl.semaphore_wait(barrier_sem, 1)

        def send_desc(i):
            src_lo = jnp.maximum(ck_s, N_ROWS - (i + 1) * BLK)
            return pltpu.make_async_remote_copy(
                src_ref=matA_ref.at[pl.ds(src_lo, BLK)],
                dst_ref=out_ref.at[pl.ds(src_lo - shift, BLK)],
                send_sem=send_sems.at[i],
                recv_sem=recv_sems.at[i],
                device_id=nbr,
                device_id_type=pl.DeviceIdType.MESH,
            )

        def keep_desc(i):
            src_lo = jnp.maximum(0, ck_s - (i + 1) * BLK)
            return pltpu.make_async_copy(
                matA_ref.at[pl.ds(src_lo, BLK)],
                out_ref.at[pl.ds(src_lo + off, BLK)],
                copy_sems.at[i],
            )

        for i in range(MAX_BLK):
            @pl.when(i < nb_s)
            def _():
                send_desc(i).start()

        for i in range(MAX_BLK):
            @pl.when(i < nbk_s)
            def _():
                keep_desc(i).start()

        for i in range(MAX_BLK):
            @pl.when(i < nbk_s)
            def _():
                keep_desc(i).wait()

        for i in range(MAX_BLK):
            @pl.when(i < nb_s)
            def _():
                send_desc(i).wait_send()

        for i in range(MAX_BLK):
            @pl.when(i < nb_s)
            def _():
                send_desc(i).wait_recv()

    out3 = pl.pallas_call(
        body,
        out_shape=jax.ShapeDtypeStruct((N_ROWS, 8, 128), jnp.bfloat16),
        in_specs=[
            pl.BlockSpec(memory_space=pltpu.SMEM),
            pl.BlockSpec(memory_space=pltpu.VMEM),
        ],
        out_specs=pl.BlockSpec(memory_space=pltpu.VMEM),
        scratch_shapes=[
            pltpu.SemaphoreType.DMA((MAX_BLK,)),
            pltpu.SemaphoreType.DMA((MAX_BLK,)),
            pltpu.SemaphoreType.DMA((MAX_BLK,)),
        ],
        compiler_params=pltpu.CompilerParams(collective_id=0),
    )(params, matA)
    return out3.reshape(N_ROWS, N_COLS)


# device time: 88194 ns/iter; 1.8531x vs baseline; 1.0295x over previous
import jax
import jax.numpy as jnp
from jax import lax
from jax.experimental import pallas as pl
from jax.experimental.pallas import tpu as pltpu

N_ROWS = 4096
N_COLS = 1024
BLK = 512
MAX_BLK = N_ROWS // BLK


def kernel(x, dest):
    zi = lax.axis_index("z")
    keep = dest == zi
    ck = jnp.sum(keep.astype(jnp.int32))
    cs = N_ROWS - ck

    order = jnp.argsort(jnp.logical_not(keep), stable=True)
    matA = x[order].astype(jnp.bfloat16).reshape(N_ROWS, 8, 128)

    nb = (cs + BLK - 1) // BLK
    nbk = (ck + BLK - 1) // BLK
    params = jnp.stack([ck, cs, nb, nbk]).astype(jnp.int32)

    def body(params_ref, matA_ref, out_ref, send_sems, recv_sems, copy_sems):
        my_x = lax.axis_index("x")
        my_y = lax.axis_index("y")
        my_z = lax.axis_index("z")
        nbr = (my_x, my_y, 1 - my_z)

        ck_s = params_ref[0]
        cs_s = params_ref[1]
        nb_s = params_ref[2]
        nbk_s = params_ref[3]
        shift = ck_s * (1 - my_z)
        off = cs_s * my_z

        barrier_sem = pltpu.get_barrier_semaphore()
        pl.semaphore_signal(
            barrier_sem, inc=1, device_id=nbr, device_id_type=pl.DeviceIdType.MESH
        )
        pl.semaphore_wait(barrier_sem, 1)

        def send_desc(i):
            src_lo = jnp.maximum(ck_s, N_ROWS - (i + 1) * BLK)
            return pltpu.make_async_remote_copy(
                src_ref=matA_ref.at[pl.ds(src_lo, BLK)],
                dst_ref=out_ref.at[pl.ds(src_lo - shift, BLK)],
                send_sem=send_sems.at[i],
                recv_sem=recv_sems.at[i],
                device_id=nbr,
                device_id_type=pl.DeviceIdType.MESH,
            )

        def keep_desc(i):
            src_lo = jnp.maximum(0, ck_s - (i + 1) * BLK)
            return pltpu.make_async_copy(
                matA_ref.at[pl.ds(src_lo, BLK)],
                out_ref.at[pl.ds(src_lo + off, BLK)],
                copy_sems.at[i],
            )

        for i in range(MAX_BLK):
            @pl.when(i < nb_s)
            def _():
                send_desc(i).start()

        for i in range(MAX_BLK):
            @pl.when(i < nbk_s)
            def _():
                keep_desc(i).start()

        for i in range(MAX_BLK):
            @pl.when(i < nbk_s)
            def _():
                keep_desc(i).wait()

        for i in range(MAX_BLK):
            @pl.when(i < nb_s)
            def _():
                send_desc(i).wait_send()

        for i in range(MAX_BLK):
            @pl.when(i < nb_s)
            def _():
                send_desc(i).wait_recv()

    out3 = pl.pallas_call(
        body,
        out_shape=jax.ShapeDtypeStruct((N_ROWS, 8, 128), jnp.bfloat16),
        in_specs=[
            pl.BlockSpec(memory_space=pltpu.SMEM),
            pl.BlockSpec(memory_space=pltpu.VMEM),
        ],
        out_specs=pl.BlockSpec(memory_space=pltpu.VMEM),
        scratch_shapes=[
            pltpu.SemaphoreType.DMA((MAX_BLK,)),
            pltpu.SemaphoreType.DMA((MAX_BLK,)),
            pltpu.SemaphoreType.DMA((MAX_BLK,)),
        ],
        compiler_params=pltpu.CompilerParams(collective_id=0),
    )(params, matA)
    return out3.reshape(N_ROWS, N_COLS)
